# Initial kernel scaffold; baseline (speedup 1.0000x reference)
import jax
import jax.numpy as jnp
from jax.experimental import pallas as pl


def kernel(h, x, e, u1, u2, u3, d2, W1, b1, g, beta, W2, b2, Wa1, ba1, Wb1, bb1, Wa2, ba2, Wb2, bb2, Wa3, ba3, Wb3, bb3):
    raise NotImplementedError("write your pallas kernel here")


# speedup vs baseline: 1.8121x; 1.8121x over previous
import functools

import jax
import jax.numpy as jnp
from jax import lax
from jax.experimental import pallas as pl
from jax.experimental.pallas import tpu as pltpu
from jax.experimental.pallas import tpu_sc as plsc

D = 128
NC = 2
NS = 16
NW = NC * NS
GRP = 128


def _cdiv(a, b):
    return (a + b - 1) // b


def _tc_precompute(h, W1a, W1b, b1):
    n = h.shape[0]

    def body(h_ref, wa_ref, wb_ref, b1_ref, p_ref, q_ref):
        hv = h_ref[...]
        p_ref[...] = jnp.dot(hv, wa_ref[...],
                             preferred_element_type=jnp.float32) + b1_ref[...]
        q_ref[...] = jnp.dot(hv, wb_ref[...],
                             preferred_element_type=jnp.float32)

    return pl.pallas_call(
        body,
        out_shape=(jax.ShapeDtypeStruct((n, D), jnp.float32),
                   jax.ShapeDtypeStruct((n, D), jnp.float32)),
    )(h, W1a, W1b, b1.reshape(1, D))


def _sc_gather(P, Q, ei2d, ej2d, e_pad):
    ng = ei2d.shape[0] // NW
    mesh = plsc.VectorSubcoreMesh(core_axis_name="c", subcore_axis_name="s")

    @functools.partial(
        pl.kernel,
        out_type=jax.ShapeDtypeStruct((e_pad, D), jnp.float32),
        mesh=mesh,
        compiler_params=pltpu.CompilerParams(use_tc_tiling_on_sc=False),
        scratch_types=[
            pltpu.VMEM((ng, GRP), jnp.int32),
            pltpu.VMEM((ng, GRP), jnp.int32),
            pltpu.VMEM((GRP, D), jnp.float32),
            pltpu.VMEM((GRP, D), jnp.float32),
            pltpu.SemaphoreType.DMA,
            pltpu.SemaphoreType.DMA,
        ],
    )
    def k(p_hbm, q_hbm, ei_hbm, ej_hbm, g_hbm, ia_v, ib_v, buf_a, buf_b,
          sem_a, sem_b):
        wid = lax.axis_index("c") * NS + lax.axis_index("s")
        rbase = wid * ng
        pltpu.sync_copy(ei_hbm.at[pl.ds(rbase, ng)], ia_v)
        pltpu.sync_copy(ej_hbm.at[pl.ds(rbase, ng)], ib_v)
        ebase = rbase * GRP

        def grp(j, carry):
            cp_a = pltpu.async_copy(p_hbm.at[ia_v.at[j]], buf_a, sem_a)
            cp_b = pltpu.async_copy(q_hbm.at[ib_v.at[j]], buf_b, sem_b)
            cp_a.wait()
            cp_b.wait()

            def row(r, c2):
                for c in range(D // 16):
                    sl = pl.ds(c * 16, 16)
                    buf_a[r, sl] = buf_a[r, sl] + buf_b[r, sl]
                return c2

            lax.fori_loop(0, GRP, row, 0)
            pltpu.sync_copy(buf_a, g_hbm.at[pl.ds(ebase + j * GRP, GRP)])
            return carry

        lax.fori_loop(0, ng, grp, 0)

    return k(P, Q, ei2d, ej2d)


def _tc_mlp(G, d2p, u1p, u2p, u3p, w1c, g, beta, W2, b2,
            Wa1, ba1, Wb1t, bb1, Wa2, ba2, Wb2t, bb2, Wa3, ba3, Wb3t, bb3,
            blk):
    e_pad = G.shape[0]
    grid = e_pad // blk

    def body(g_ref, d2_ref, u1_ref, u2_ref, u3_ref, w1c_ref, gam_ref,
             beta_ref, w2_ref, b2_ref, wa1_ref, ba1_ref, wb1_ref, bb1_ref,
             wa2_ref, ba2_ref, wb2_ref, bb2_ref, wa3_ref, ba3_ref, wb3_ref,
             bb3_ref, out_ref):
        t = g_ref[...] + d2_ref[...] * w1c_ref[...]
        a = t * jax.nn.sigmoid(t)
        mu = jnp.mean(a, axis=-1, keepdims=True)
        var = jnp.mean((a - mu) ** 2, axis=-1, keepdims=True)
        an = (a - mu) * lax.rsqrt(var + 1e-5) * gam_ref[...] + beta_ref[...]
        m2 = jnp.dot(an, w2_ref[...], preferred_element_type=jnp.float32)
        m2 = m2 + b2_ref[...]
        m2 = m2 * jax.nn.sigmoid(m2)

        def head(wa_ref, ba_ref, wb_ref, bb_ref):
            q = jnp.dot(m2, wa_ref[...], preferred_element_type=jnp.float32)
            q = q + ba_ref[...]
            q = q * jax.nn.sigmoid(q)
            return (jnp.sum(q * wb_ref[...], axis=-1, keepdims=True)
                    + bb_ref[...])

        s1 = head(wa1_ref, ba1_ref, wb1_ref, bb1_ref)
        s2 = head(wa2_ref, ba2_ref, wb2_ref, bb2_ref)
        s3 = head(wa3_ref, ba3_ref, wb3_ref, bb3_ref)
        out_ref[...] = u1_ref[...] * s1 + u2_ref[...] * s2 + u3_ref[...] * s3

    edge_spec = lambda w: pl.BlockSpec((blk, w), lambda i: (i, 0))
    full = lambda shp: pl.BlockSpec(shp, lambda i: (0,) * len(shp))
    return pl.pallas_call(
        body,
        grid=(grid,),
        in_specs=[
            edge_spec(D), edge_spec(1), edge_spec(8), edge_spec(8),
            edge_spec(8),
            full((1, D)), full((1, D)), full((1, D)),
            full((D, D)), full((1, D)),
            full((D, D)), full((1, D)), full((1, D)), full((1, 1)),
            full((D, D)), full((1, D)), full((1, D)), full((1, 1)),
            full((D, D)), full((1, D)), full((1, D)), full((1, 1)),
        ],
        out_specs=edge_spec(8),
        out_shape=jax.ShapeDtypeStruct((e_pad, 8), jnp.float32),
    )(G, d2p, u1p, u2p, u3p, w1c.reshape(1, D), g.reshape(1, D),
      beta.reshape(1, D), W2, b2.reshape(1, D),
      Wa1, ba1.reshape(1, D), Wb1t, bb1.reshape(1, 1),
      Wa2, ba2.reshape(1, D), Wb2t, bb2.reshape(1, 1),
      Wa3, ba3.reshape(1, D), Wb3t, bb3.reshape(1, 1))


def _sc_scatter(msg, ei2d, init, nacc):
    ng = ei2d.shape[0] // NW
    ew = ng * GRP
    mesh = plsc.VectorSubcoreMesh(core_axis_name="c", subcore_axis_name="s")

    @functools.partial(
        pl.kernel,
        out_type=jax.ShapeDtypeStruct((NC, nacc, 8), jnp.float32),
        mesh=mesh,
        compiler_params=pltpu.CompilerParams(use_tc_tiling_on_sc=False),
        scratch_types=[
            pltpu.VMEM((ew, 8), jnp.float32),
            pltpu.VMEM((ng, GRP), jnp.int32),
            pltpu.VMEM_SHARED((nacc, 8), jnp.float32),
        ],
    )
    def k(msg_hbm, ei_hbm, init_hbm, out_hbm, msg_v, idx_v, acc_sh):
        cid = lax.axis_index("c")
        sid = lax.axis_index("s")
        wid = cid * NS + sid
        rbase = wid * ng

        @pl.when(sid == 0)
        def _():
            pltpu.sync_copy(init_hbm.at[cid], acc_sh)

        pltpu.sync_copy(msg_hbm.at[pl.ds(rbase * GRP, ew)], msg_v)
        pltpu.sync_copy(ei_hbm.at[pl.ds(rbase, ng)], idx_v)
        plsc.subcore_barrier()

        def grp(j, carry):
            pltpu.sync_copy(msg_v.at[pl.ds(j * GRP, GRP)],
                            acc_sh.at[idx_v.at[j]], add=True)
            return carry

        lax.fori_loop(0, ng, grp, 0)
        plsc.subcore_barrier()

        @pl.when(sid == 0)
        def _():
            pltpu.sync_copy(acc_sh, out_hbm.at[cid])

    return k(msg, ei2d, init)


def kernel(h, x, e, u1, u2, u3, d2, W1, b1, g, beta, W2, b2,
           Wa1, ba1, Wb1, bb1, Wa2, ba2, Wb2, bb2, Wa3, ba3, Wb3, bb3):
    n = h.shape[0]
    e_num = e.shape[1]
    e_pad = _cdiv(e_num, NW * GRP * 8) * NW * GRP * 8
    pad = e_pad - e_num
    nacc = _cdiv(n + 1, NS) * NS

    ei = e[0].astype(jnp.int32)
    ej = e[1].astype(jnp.int32)
    ei_p = jnp.concatenate([ei, jnp.full((pad,), n, jnp.int32)])
    ej_p = jnp.concatenate([ej, jnp.zeros((pad,), jnp.int32)])
    ei_g = jnp.concatenate([ei, jnp.zeros((pad,), jnp.int32)])
    ei2d_s = ei_p.reshape(e_pad // GRP, GRP)
    ei2d_g = ei_g.reshape(e_pad // GRP, GRP)
    ej2d = ej_p.reshape(e_pad // GRP, GRP)

    zpad8 = jnp.zeros((pad, 8), jnp.float32)
    u1p = jnp.concatenate([jnp.pad(u1, ((0, 0), (0, 5))), zpad8])
    u2p = jnp.concatenate([jnp.pad(u2, ((0, 0), (0, 5))), zpad8])
    u3p = jnp.concatenate([jnp.pad(u3, ((0, 0), (0, 5))), zpad8])
    d2p = jnp.concatenate([d2, jnp.zeros((pad, 1), jnp.float32)])

    W1a = W1[:D]
    W1b = W1[D:2 * D]
    w1c = W1[2 * D]

    P, Q = _tc_precompute(h, W1a, W1b, b1)
    G = _sc_gather(P, Q, ei2d_g, ej2d, e_pad)
    msg = _tc_mlp(G, d2p, u1p, u2p, u3p, w1c, g, beta, W2, b2,
                  Wa1, ba1, Wb1.reshape(1, D), bb1,
                  Wa2, ba2, Wb2.reshape(1, D), bb2,
                  Wa3, ba3, Wb3.reshape(1, D), bb3, blk=2048)

    xpad = jnp.zeros((NC, nacc, 8), jnp.float32)
    xpad = xpad.at[0, :n, :3].set(x)
    partials = _sc_scatter(msg, ei2d_s, xpad, nacc)
    return partials[0, :n, :3] + partials[1, :n, :3]


# speedup vs baseline: 1.8314x; 1.0106x over previous
import functools

import jax
import jax.numpy as jnp
from jax import lax
from jax.experimental import pallas as pl
from jax.experimental.pallas import tpu as pltpu
from jax.experimental.pallas import tpu_sc as plsc

D = 128
NC = 2
NS = 16
NW = NC * NS
GRP = 128


def _cdiv(a, b):
    return (a + b - 1) // b


def _tc_precompute(h, W1a, W1b, b1):
    n = h.shape[0]

    def body(h_ref, wa_ref, wb_ref, b1_ref, p_ref, q_ref):
        hv = h_ref[...]
        p_ref[...] = jnp.dot(hv, wa_ref[...],
                             preferred_element_type=jnp.float32) + b1_ref[...]
        q_ref[...] = jnp.dot(hv, wb_ref[...],
                             preferred_element_type=jnp.float32)

    return pl.pallas_call(
        body,
        out_shape=(jax.ShapeDtypeStruct((n, D), jnp.float32),
                   jax.ShapeDtypeStruct((n, D), jnp.float32)),
    )(h, W1a, W1b, b1.reshape(1, D))


def _sc_gather(P, Q, ei2d, ej2d, e_pad):
    ng = ei2d.shape[0] // NW
    mesh = plsc.VectorSubcoreMesh(core_axis_name="c", subcore_axis_name="s")

    @functools.partial(
        pl.kernel,
        out_type=jax.ShapeDtypeStruct((e_pad, D), jnp.float32),
        mesh=mesh,
        compiler_params=pltpu.CompilerParams(use_tc_tiling_on_sc=False),
        scratch_types=[
            pltpu.VMEM((ng, GRP), jnp.int32),
            pltpu.VMEM((ng, GRP), jnp.int32),
            pltpu.VMEM((GRP, D), jnp.float32),
            pltpu.VMEM((GRP, D), jnp.float32),
            pltpu.VMEM((GRP, D), jnp.float32),
            pltpu.VMEM((GRP, D), jnp.float32),
            pltpu.VMEM((GRP, D), jnp.float32),
            pltpu.VMEM((GRP, D), jnp.float32),
            pltpu.SemaphoreType.DMA,
            pltpu.SemaphoreType.DMA,
            pltpu.SemaphoreType.DMA,
            pltpu.SemaphoreType.DMA,
            pltpu.SemaphoreType.DMA,
            pltpu.SemaphoreType.DMA,
        ],
    )
    def k(p_hbm, q_hbm, ei_hbm, ej_hbm, g_hbm, ia_v, ib_v,
          ba0, ba1, bb0, bb1, bo0, bo1,
          sa0, sa1, sb0, sb1, sw0, sw1):
        wid = lax.axis_index("c") * NS + lax.axis_index("s")
        rbase = wid * ng
        pltpu.sync_copy(ei_hbm.at[pl.ds(rbase, ng)], ia_v)
        pltpu.sync_copy(ej_hbm.at[pl.ds(rbase, ng)], ib_v)
        ebase = rbase * GRP
        bufs_a = (ba0, ba1)
        bufs_b = (bb0, bb1)
        bufs_o = (bo0, bo1)
        sems_a = (sa0, sa1)
        sems_b = (sb0, sb1)
        sems_w = (sw0, sw1)

        for b in range(2):
            pltpu.async_copy(p_hbm.at[ia_v.at[b]], bufs_a[b], sems_a[b])
            pltpu.async_copy(q_hbm.at[ib_v.at[b]], bufs_b[b], sems_b[b])

        def j2_loop(j2, carry):
            for b in range(2):
                j = j2 * 2 + b
                pltpu.make_async_copy(p_hbm.at[ia_v.at[j]], bufs_a[b],
                                      sems_a[b]).wait()
                pltpu.make_async_copy(q_hbm.at[ib_v.at[j]], bufs_b[b],
                                      sems_b[b]).wait()
                @pl.when(j2 > 0)
                def _():
                    pltpu.make_async_copy(
                        bufs_o[b], g_hbm.at[pl.ds(ebase, GRP)],
                        sems_w[b]).wait()

                def row(r, c2):
                    for c in range(D // 16):
                        sl = pl.ds(c * 16, 16)
                        bufs_o[b][r, sl] = bufs_a[b][r, sl] + bufs_b[b][r, sl]
                    return c2

                lax.fori_loop(0, GRP, row, 0)
                pltpu.async_copy(bufs_o[b],
                                 g_hbm.at[pl.ds(ebase + j * GRP, GRP)],
                                 sems_w[b])
                @pl.when(j + 2 < ng)
                def _():
                    pltpu.async_copy(p_hbm.at[ia_v.at[j + 2]], bufs_a[b],
                                     sems_a[b])
                    pltpu.async_copy(q_hbm.at[ib_v.at[j + 2]], bufs_b[b],
                                     sems_b[b])
            return carry

        lax.fori_loop(0, ng // 2, j2_loop, 0)
        for b in range(2):
            pltpu.make_async_copy(bufs_o[b], g_hbm.at[pl.ds(ebase, GRP)],
                                  sems_w[b]).wait()

    return k(P, Q, ei2d, ej2d)


def _tc_mlp(G, d2p, u1p, u2p, u3p, w1c, g, beta, W2, b2,
            Wa1, ba1, Wb1t, bb1, Wa2, ba2, Wb2t, bb2, Wa3, ba3, Wb3t, bb3,
            blk):
    e_pad = G.shape[0]
    grid = e_pad // blk

    def body(g_ref, d2_ref, u1_ref, u2_ref, u3_ref, w1c_ref, gam_ref,
             beta_ref, w2_ref, b2_ref, wa1_ref, ba1_ref, wb1_ref, bb1_ref,
             wa2_ref, ba2_ref, wb2_ref, bb2_ref, wa3_ref, ba3_ref, wb3_ref,
             bb3_ref, out_ref):
        t = g_ref[...] + d2_ref[...] * w1c_ref[...]
        a = t * jax.nn.sigmoid(t)
        mu = jnp.mean(a, axis=-1, keepdims=True)
        var = jnp.mean((a - mu) ** 2, axis=-1, keepdims=True)
        an = (a - mu) * lax.rsqrt(var + 1e-5) * gam_ref[...] + beta_ref[...]
        m2 = jnp.dot(an, w2_ref[...], preferred_element_type=jnp.float32)
        m2 = m2 + b2_ref[...]
        m2 = m2 * jax.nn.sigmoid(m2)

        def head(wa_ref, ba_ref, wb_ref, bb_ref):
            q = jnp.dot(m2, wa_ref[...], preferred_element_type=jnp.float32)
            q = q + ba_ref[...]
            q = q * jax.nn.sigmoid(q)
            return (jnp.sum(q * wb_ref[...], axis=-1, keepdims=True)
                    + bb_ref[...])

        s1 = head(wa1_ref, ba1_ref, wb1_ref, bb1_ref)
        s2 = head(wa2_ref, ba2_ref, wb2_ref, bb2_ref)
        s3 = head(wa3_ref, ba3_ref, wb3_ref, bb3_ref)
        out_ref[...] = u1_ref[...] * s1 + u2_ref[...] * s2 + u3_ref[...] * s3

    edge_spec = lambda w: pl.BlockSpec((blk, w), lambda i: (i, 0))
    full = lambda shp: pl.BlockSpec(shp, lambda i: (0,) * len(shp))
    return pl.pallas_call(
        body,
        grid=(grid,),
        in_specs=[
            edge_spec(D), edge_spec(1), edge_spec(8), edge_spec(8),
            edge_spec(8),
            full((1, D)), full((1, D)), full((1, D)),
            full((D, D)), full((1, D)),
            full((D, D)), full((1, D)), full((1, D)), full((1, 1)),
            full((D, D)), full((1, D)), full((1, D)), full((1, 1)),
            full((D, D)), full((1, D)), full((1, D)), full((1, 1)),
        ],
        out_specs=edge_spec(8),
        out_shape=jax.ShapeDtypeStruct((e_pad, 8), jnp.float32),
    )(G, d2p, u1p, u2p, u3p, w1c.reshape(1, D), g.reshape(1, D),
      beta.reshape(1, D), W2, b2.reshape(1, D),
      Wa1, ba1.reshape(1, D), Wb1t, bb1.reshape(1, 1),
      Wa2, ba2.reshape(1, D), Wb2t, bb2.reshape(1, 1),
      Wa3, ba3.reshape(1, D), Wb3t, bb3.reshape(1, 1))


def _sc_scatter(msg, ei2d, init, nacc):
    ng = ei2d.shape[0] // NW
    ew = ng * GRP
    mesh = plsc.VectorSubcoreMesh(core_axis_name="c", subcore_axis_name="s")

    @functools.partial(
        pl.kernel,
        out_type=jax.ShapeDtypeStruct((NC, nacc, 8), jnp.float32),
        mesh=mesh,
        compiler_params=pltpu.CompilerParams(use_tc_tiling_on_sc=False),
        scratch_types=[
            pltpu.VMEM((ew, 8), jnp.float32),
            pltpu.VMEM((ng, GRP), jnp.int32),
            pltpu.VMEM_SHARED((nacc, 8), jnp.float32),
        ],
    )
    def k(msg_hbm, ei_hbm, init_hbm, out_hbm, msg_v, idx_v, acc_sh):
        cid = lax.axis_index("c")
        sid = lax.axis_index("s")
        wid = cid * NS + sid
        rbase = wid * ng

        @pl.when(sid == 0)
        def _():
            pltpu.sync_copy(init_hbm.at[cid], acc_sh)

        pltpu.sync_copy(msg_hbm.at[pl.ds(rbase * GRP, ew)], msg_v)
        pltpu.sync_copy(ei_hbm.at[pl.ds(rbase, ng)], idx_v)
        plsc.subcore_barrier()

        def grp(j, carry):
            pltpu.sync_copy(msg_v.at[pl.ds(j * GRP, GRP)],
                            acc_sh.at[idx_v.at[j]], add=True)
            return carry

        lax.fori_loop(0, ng, grp, 0)
        plsc.subcore_barrier()

        @pl.when(sid == 0)
        def _():
            pltpu.sync_copy(acc_sh, out_hbm.at[cid])

    return k(msg, ei2d, init)


def kernel(h, x, e, u1, u2, u3, d2, W1, b1, g, beta, W2, b2,
           Wa1, ba1, Wb1, bb1, Wa2, ba2, Wb2, bb2, Wa3, ba3, Wb3, bb3):
    n = h.shape[0]
    e_num = e.shape[1]
    e_pad = _cdiv(e_num, NW * GRP * 8) * NW * GRP * 8
    pad = e_pad - e_num
    nacc = _cdiv(n + 1, NS) * NS

    ei = e[0].astype(jnp.int32)
    ej = e[1].astype(jnp.int32)
    ei_p = jnp.concatenate([ei, jnp.full((pad,), n, jnp.int32)])
    ej_p = jnp.concatenate([ej, jnp.zeros((pad,), jnp.int32)])
    ei_g = jnp.concatenate([ei, jnp.zeros((pad,), jnp.int32)])
    ei2d_s = ei_p.reshape(e_pad // GRP, GRP)
    ei2d_g = ei_g.reshape(e_pad // GRP, GRP)
    ej2d = ej_p.reshape(e_pad // GRP, GRP)

    zpad8 = jnp.zeros((pad, 8), jnp.float32)
    u1p = jnp.concatenate([jnp.pad(u1, ((0, 0), (0, 5))), zpad8])
    u2p = jnp.concatenate([jnp.pad(u2, ((0, 0), (0, 5))), zpad8])
    u3p = jnp.concatenate([jnp.pad(u3, ((0, 0), (0, 5))), zpad8])
    d2p = jnp.concatenate([d2, jnp.zeros((pad, 1), jnp.float32)])

    W1a = W1[:D]
    W1b = W1[D:2 * D]
    w1c = W1[2 * D]

    P, Q = _tc_precompute(h, W1a, W1b, b1)
    G = _sc_gather(P, Q, ei2d_g, ej2d, e_pad)
    msg = _tc_mlp(G, d2p, u1p, u2p, u3p, w1c, g, beta, W2, b2,
                  Wa1, ba1, Wb1.reshape(1, D), bb1,
                  Wa2, ba2, Wb2.reshape(1, D), bb2,
                  Wa3, ba3, Wb3.reshape(1, D), bb3, blk=2048)

    xpad = jnp.zeros((NC, nacc, 8), jnp.float32)
    xpad = xpad.at[0, :n, :3].set(x)
    partials = _sc_scatter(msg, ei2d_s, xpad, nacc)
    return partials[0, :n, :3] + partials[1, :n, :3]


# speedup vs baseline: 2.3879x; 1.3039x over previous
import functools

import jax
import jax.numpy as jnp
from jax import lax
from jax.experimental import pallas as pl
from jax.experimental.pallas import tpu as pltpu
from jax.experimental.pallas import tpu_sc as plsc

D = 128
NC = 2
NS = 16
NW = NC * NS
GRP = 128


def _cdiv(a, b):
    return (a + b - 1) // b


def _tc_precompute(h, W1a, W1b, b1):
    n = h.shape[0]

    def body(h_ref, wa_ref, wb_ref, b1_ref, p_ref, q_ref):
        hv = h_ref[...]
        p_ref[...] = jnp.dot(hv, wa_ref[...],
                             preferred_element_type=jnp.float32) + b1_ref[...]
        q_ref[...] = jnp.dot(hv, wb_ref[...],
                             preferred_element_type=jnp.float32)

    return pl.pallas_call(
        body,
        out_shape=(jax.ShapeDtypeStruct((n, D), jnp.float32),
                   jax.ShapeDtypeStruct((n, D), jnp.float32)),
    )(h, W1a, W1b, b1.reshape(1, D))


def _sc_gather(P, Q, ei2d, ej2d, e_pad):
    ng = ei2d.shape[0] // NW
    mesh = plsc.VectorSubcoreMesh(core_axis_name="c", subcore_axis_name="s")

    @functools.partial(
        pl.kernel,
        out_type=jax.ShapeDtypeStruct((e_pad, D), jnp.float32),
        mesh=mesh,
        compiler_params=pltpu.CompilerParams(use_tc_tiling_on_sc=False),
        scratch_types=[
            pltpu.VMEM((ng, GRP), jnp.int32),
            pltpu.VMEM((ng, GRP), jnp.int32),
            pltpu.VMEM((GRP, D), jnp.float32),
            pltpu.VMEM((GRP, D), jnp.float32),
            pltpu.VMEM((GRP, D), jnp.float32),
            pltpu.VMEM((GRP, D), jnp.float32),
            pltpu.VMEM((GRP, D), jnp.float32),
            pltpu.VMEM((GRP, D), jnp.float32),
            pltpu.SemaphoreType.DMA,
            pltpu.SemaphoreType.DMA,
            pltpu.SemaphoreType.DMA,
            pltpu.SemaphoreType.DMA,
            pltpu.SemaphoreType.DMA,
            pltpu.SemaphoreType.DMA,
        ],
    )
    def k(p_hbm, q_hbm, ei_hbm, ej_hbm, g_hbm, ia_v, ib_v,
          ba0, ba1, bb0, bb1, bo0, bo1,
          sa0, sa1, sb0, sb1, sw0, sw1):
        wid = lax.axis_index("c") * NS + lax.axis_index("s")
        rbase = wid * ng
        pltpu.sync_copy(ei_hbm.at[pl.ds(rbase, ng)], ia_v)
        pltpu.sync_copy(ej_hbm.at[pl.ds(rbase, ng)], ib_v)
        ebase = rbase * GRP
        bufs_a = (ba0, ba1)
        bufs_b = (bb0, bb1)
        bufs_o = (bo0, bo1)
        sems_a = (sa0, sa1)
        sems_b = (sb0, sb1)
        sems_w = (sw0, sw1)

        for b in range(2):
            pltpu.async_copy(p_hbm.at[ia_v.at[b]], bufs_a[b], sems_a[b])
            pltpu.async_copy(q_hbm.at[ib_v.at[b]], bufs_b[b], sems_b[b])

        def j2_loop(j2, carry):
            for b in range(2):
                j = j2 * 2 + b
                pltpu.make_async_copy(p_hbm.at[ia_v.at[j]], bufs_a[b],
                                      sems_a[b]).wait()
                pltpu.make_async_copy(q_hbm.at[ib_v.at[j]], bufs_b[b],
                                      sems_b[b]).wait()
                @pl.when(j2 > 0)
                def _():
                    pltpu.make_async_copy(
                        bufs_o[b], g_hbm.at[pl.ds(ebase, GRP)],
                        sems_w[b]).wait()

                def row(r4, c2):
                    for dr in range(4):
                        r = r4 * 4 + dr
                        for c in range(D // 16):
                            sl = pl.ds(c * 16, 16)
                            bufs_o[b][r, sl] = (bufs_a[b][r, sl]
                                                + bufs_b[b][r, sl])
                    return c2

                lax.fori_loop(0, GRP // 4, row, 0)
                pltpu.async_copy(bufs_o[b],
                                 g_hbm.at[pl.ds(ebase + j * GRP, GRP)],
                                 sems_w[b])
                @pl.when(j + 2 < ng)
                def _():
                    pltpu.async_copy(p_hbm.at[ia_v.at[j + 2]], bufs_a[b],
                                     sems_a[b])
                    pltpu.async_copy(q_hbm.at[ib_v.at[j + 2]], bufs_b[b],
                                     sems_b[b])
            return carry

        lax.fori_loop(0, ng // 2, j2_loop, 0)
        for b in range(2):
            pltpu.make_async_copy(bufs_o[b], g_hbm.at[pl.ds(ebase, GRP)],
                                  sems_w[b]).wait()

    return k(P, Q, ei2d, ej2d)


def _tc_mlp(G, d2p, u1p, u2p, u3p, w1c, g, beta, W2, b2,
            WaCat, baCat, WbBd, bbRow, blk):
    e_pad = G.shape[0]
    grid = e_pad // blk

    n_real = d2p.shape[0] // blk
    def body(g_ref, d2_ref, u1_ref, u2_ref, u3_ref, w1c_ref, gam_ref,
             beta_ref, w2_ref, b2_ref, wa_ref, ba_ref, wb_ref, bb_ref,
             out_ref):
        t = g_ref[...] + d2_ref[...] * w1c_ref[...]
        a = t * jax.nn.sigmoid(t)
        cone = jnp.full((D, 1), 1.0 / D, jnp.float32)
        mu = jnp.dot(a, cone, preferred_element_type=jnp.float32)
        msq = jnp.dot(a * a, cone, preferred_element_type=jnp.float32)
        var = msq - mu * mu
        an = (a - mu) * lax.rsqrt(var + 1e-5) * gam_ref[...] + beta_ref[...]
        m2 = jnp.dot(an, w2_ref[...], preferred_element_type=jnp.float32)
        m2 = m2 + b2_ref[...]
        m2 = m2 * jax.nn.sigmoid(m2)
        q = jnp.dot(m2, wa_ref[...], preferred_element_type=jnp.float32)
        q = q + ba_ref[...]
        q = q * jax.nn.sigmoid(q)
        s = jnp.dot(q, wb_ref[...], preferred_element_type=jnp.float32)
        s = s + bb_ref[...]
        m3 = (u1_ref[...] * s[:, 0:1] + u2_ref[...] * s[:, 1:2]
              + u3_ref[...] * s[:, 2:3])
        out_ref[...] = jnp.concatenate(
            [m3, jnp.zeros((blk, 5), jnp.float32)], axis=-1)

    edge_spec = lambda w: pl.BlockSpec((blk, w), lambda i: (i, 0))
    clamped = lambda w: pl.BlockSpec(
        (blk, w), lambda i: (jnp.minimum(i, n_real - 1), 0))
    full = lambda shp: pl.BlockSpec(shp, lambda i: (0,) * len(shp))
    return pl.pallas_call(
        body,
        grid=(grid,),
        in_specs=[
            edge_spec(D), clamped(1), clamped(3), clamped(3), clamped(3),
            full((1, D)), full((1, D)), full((1, D)),
            full((D, D)), full((1, D)),
            full((D, 3 * D)), full((1, 3 * D)),
            full((3 * D, 8)), full((1, 8)),
        ],
        out_specs=edge_spec(8),
        out_shape=jax.ShapeDtypeStruct((e_pad, 8), jnp.float32),
    )(G, d2p, u1p, u2p, u3p, w1c.reshape(1, D), g.reshape(1, D),
      beta.reshape(1, D), W2, b2.reshape(1, D),
      WaCat, baCat, WbBd, bbRow)


def _sc_scatter(msg, ei2d, init, nacc):
    ng = ei2d.shape[0] // NW
    ew = ng * GRP
    mesh = plsc.VectorSubcoreMesh(core_axis_name="c", subcore_axis_name="s")

    @functools.partial(
        pl.kernel,
        out_type=jax.ShapeDtypeStruct((NC, nacc, 8), jnp.float32),
        mesh=mesh,
        compiler_params=pltpu.CompilerParams(use_tc_tiling_on_sc=False),
        scratch_types=[
            pltpu.VMEM((ew, 8), jnp.float32),
            pltpu.VMEM((ng, GRP), jnp.int32),
            pltpu.VMEM_SHARED((nacc, 8), jnp.float32),
        ],
    )
    def k(msg_hbm, ei_hbm, init_hbm, out_hbm, msg_v, idx_v, acc_sh):
        cid = lax.axis_index("c")
        sid = lax.axis_index("s")
        wid = cid * NS + sid
        rbase = wid * ng

        @pl.when(sid == 0)
        def _():
            pltpu.sync_copy(init_hbm.at[cid], acc_sh)

        pltpu.sync_copy(msg_hbm.at[pl.ds(rbase * GRP, ew)], msg_v)
        pltpu.sync_copy(ei_hbm.at[pl.ds(rbase, ng)], idx_v)
        plsc.subcore_barrier()

        def grp(j, carry):
            pltpu.sync_copy(msg_v.at[pl.ds(j * GRP, GRP)],
                            acc_sh.at[idx_v.at[j]], add=True)
            return carry

        lax.fori_loop(0, ng, grp, 0)
        plsc.subcore_barrier()

        @pl.when(sid == 0)
        def _():
            pltpu.sync_copy(acc_sh, out_hbm.at[cid])

    return k(msg, ei2d, init)


def kernel(h, x, e, u1, u2, u3, d2, W1, b1, g, beta, W2, b2,
           Wa1, ba1, Wb1, bb1, Wa2, ba2, Wb2, bb2, Wa3, ba3, Wb3, bb3):
    n = h.shape[0]
    e_num = e.shape[1]
    e_pad = _cdiv(e_num, NW * GRP * 8) * NW * GRP * 8
    pad = e_pad - e_num
    nacc = _cdiv(n + 1, NS) * NS

    ei = e[0].astype(jnp.int32)
    ej = e[1].astype(jnp.int32)
    ei_p = jnp.concatenate([ei, jnp.full((pad,), n, jnp.int32)])
    ej_p = jnp.concatenate([ej, jnp.zeros((pad,), jnp.int32)])
    ei_g = jnp.concatenate([ei, jnp.zeros((pad,), jnp.int32)])
    ei2d_s = ei_p.reshape(e_pad // GRP, GRP)
    ei2d_g = ei_g.reshape(e_pad // GRP, GRP)
    ej2d = ej_p.reshape(e_pad // GRP, GRP)

    blk = 2560
    if e_num % blk or e_pad % blk:
        blk = 0
        for cand in (2048, 1280, 1024, 640, 512, 256, 128):
            if e_num % cand == 0 and e_pad % cand == 0:
                blk = cand
                break
        assert blk, "no common block size for edge MLP"

    W1a = W1[:D]
    W1b = W1[D:2 * D]
    w1c = W1[2 * D]

    P, Q = _tc_precompute(h, W1a, W1b, b1)
    G = _sc_gather(P, Q, ei2d_g, ej2d, e_pad)
    WaCat = jnp.concatenate([Wa1, Wa2, Wa3], axis=1)
    baCat = jnp.concatenate([ba1, ba2, ba3]).reshape(1, 3 * D)
    WbBd = jnp.zeros((3 * D, 8), jnp.float32)
    WbBd = WbBd.at[0:D, 0].set(Wb1[:, 0])
    WbBd = WbBd.at[D:2 * D, 1].set(Wb2[:, 0])
    WbBd = WbBd.at[2 * D:3 * D, 2].set(Wb3[:, 0])
    bbRow = jnp.zeros((1, 8), jnp.float32)
    bbRow = bbRow.at[0, 0].set(bb1[0]).at[0, 1].set(bb2[0]).at[0, 2].set(bb3[0])
    msg = _tc_mlp(G, d2, u1, u2, u3, w1c, g, beta, W2, b2,
                  WaCat, baCat, WbBd, bbRow, blk=blk)
    xpad = jnp.zeros((NC, nacc, 8), jnp.float32)
    xpad = xpad.at[0, :n, :3].set(x)
    partials = _sc_scatter(msg, ei2d_s, xpad, nacc)
    return partials[0, :n, :3] + partials[1, :n, :3]
